# Initial kernel scaffold; baseline (speedup 1.0000x reference)
import jax
import jax.numpy as jnp
from jax.experimental import pallas as pl


def kernel(sentence, table):
    raise NotImplementedError("write your pallas kernel here")


# speedup vs baseline: 11.4787x; 11.4787x over previous
import functools

import jax
import jax.numpy as jnp
from jax import lax
from jax.experimental import pallas as pl
from jax.experimental.pallas import tpu as pltpu
from jax.experimental.pallas import tpu_sc as plsc

VOCAB = 100000
D = 128
S = 200
B = 4096

NC = 2
NS = 16
L = 16
BT = B // (NC * NS)
SC_B = B // NC


def _mean_embed(sentence, table):
    mesh = plsc.VectorSubcoreMesh(core_axis_name="c", subcore_axis_name="s")

    @functools.partial(
        pl.kernel,
        mesh=mesh,
        out_type=jax.ShapeDtypeStruct((B, D), jnp.float32),
        scratch_types=[
            pltpu.VMEM((S, BT), jnp.int32),
            pltpu.VMEM((2, BT, D), jnp.float32),
            pltpu.VMEM((BT,), jnp.int32),
            pltpu.VMEM((BT, D), jnp.float32),
            pltpu.VMEM_SHARED((SC_B, D), jnp.float32),
            pltpu.SemaphoreType.DMA,
            pltpu.SemaphoreType.DMA,
        ],
    )
    def k(sent_hbm, table_hbm, out_hbm, idx_v, rows_v, dst_v, acc_v,
          accum_sh, sem0, sem1):
        cid = lax.axis_index("c")
        sid = lax.axis_index("s")
        tid = cid * NS + sid
        gbase = tid * BT
        lbase = sid * BT

        pltpu.sync_copy(sent_hbm.at[:, pl.ds(gbase, BT)], idx_v)

        for j in range(BT // L):
            dst_v[pl.ds(j * L, L)] = (
                jnp.full((L,), lbase + j * L, jnp.int32)
                + lax.iota(jnp.int32, L)
            )

        zeros = jnp.zeros((L,), jnp.float32)

        def zbody(r, carry):
            for j in range(D // L):
                acc_v[r, pl.ds(j * L, L)] = zeros
            return carry

        lax.fori_loop(0, BT, zbody, 0)
        pltpu.sync_copy(acc_v, accum_sh.at[pl.ds(lbase, BT)])

        sems = (sem0, sem1)

        pltpu.async_copy(table_hbm.at[idx_v.at[0]], rows_v.at[0], sem0)
        pltpu.async_copy(table_hbm.at[idx_v.at[1]], rows_v.at[1], sem1)

        def body(g, carry):
            for b in range(2):
                s = 2 * g + b
                pltpu.make_async_copy(
                    table_hbm.at[idx_v.at[0]], rows_v.at[b], sems[b]
                ).wait()
                pltpu.sync_copy(rows_v.at[b], accum_sh.at[dst_v], add=True)

                @pl.when(s + 2 < S)
                def _refill():
                    pltpu.async_copy(
                        table_hbm.at[idx_v.at[s + 2]], rows_v.at[b], sems[b]
                    )
            return carry

        lax.fori_loop(0, S // 2, body, 0)

        pltpu.sync_copy(accum_sh.at[pl.ds(lbase, BT)], acc_v)
        inv = jnp.full((L,), 1.0 / S, jnp.float32)

        def sbody(r, carry):
            for j in range(D // L):
                acc_v[r, pl.ds(j * L, L)] = acc_v[r, pl.ds(j * L, L)] * inv
            return carry

        lax.fori_loop(0, BT, sbody, 0)
        pltpu.sync_copy(acc_v, out_hbm.at[pl.ds(gbase, BT)])

    return k(sentence, table)


def kernel(sentence, table):
    return _mean_embed(sentence, table)


# speedup vs baseline: 13.2458x; 1.1539x over previous
import functools

import jax
import jax.numpy as jnp
from jax import lax
from jax.experimental import pallas as pl
from jax.experimental.pallas import tpu as pltpu
from jax.experimental.pallas import tpu_sc as plsc

VOCAB = 100000
D = 128
S = 200
B = 4096

NC = 2
NS = 16
L = 16
BT = B // (NC * NS)
SC_B = B // NC


def _mean_embed(sentence, table):
    mesh = plsc.VectorSubcoreMesh(core_axis_name="c", subcore_axis_name="s")

    @functools.partial(
        pl.kernel,
        mesh=mesh,
        out_type=jax.ShapeDtypeStruct((B, D), jnp.float32),
        scratch_types=[
            pltpu.VMEM((S, BT), jnp.int32),
            pltpu.VMEM((4, BT, D), jnp.float32),
            pltpu.VMEM((BT,), jnp.int32),
            pltpu.VMEM((BT, D), jnp.float32),
            pltpu.VMEM_SHARED((SC_B, D), jnp.float32),
            pltpu.SemaphoreType.DMA,
            pltpu.SemaphoreType.DMA,
            pltpu.SemaphoreType.DMA,
            pltpu.SemaphoreType.DMA,
        ],
    )
    def k(sent_hbm, table_hbm, out_hbm, idx_v, rows_v, dst_v, acc_v,
          accum_sh, sem0, sem1, sem2, sem3):
        cid = lax.axis_index("c")
        sid = lax.axis_index("s")
        tid = cid * NS + sid
        gbase = tid * BT
        lbase = sid * BT

        pltpu.sync_copy(sent_hbm.at[:, pl.ds(gbase, BT)], idx_v)

        for j in range(BT // L):
            dst_v[pl.ds(j * L, L)] = (
                jnp.full((L,), lbase + j * L, jnp.int32)
                + lax.iota(jnp.int32, L)
            )

        sems = (sem0, sem1, sem2, sem3)
        NB = 4

        for b in range(NB):
            pltpu.async_copy(table_hbm.at[idx_v.at[b]], rows_v.at[b], sems[b])

        pltpu.make_async_copy(
            table_hbm.at[idx_v.at[0]], rows_v.at[0], sems[0]
        ).wait()
        pltpu.sync_copy(rows_v.at[0], accum_sh.at[dst_v])
        pltpu.async_copy(table_hbm.at[idx_v.at[NB]], rows_v.at[0], sems[0])

        def tail_body(g, carry):
            for b in range(NB):
                s = NB * g + b + 1
                bb = (b + 1) % NB

                @pl.when(s < S)
                def _step():
                    pltpu.make_async_copy(
                        table_hbm.at[idx_v.at[0]], rows_v.at[bb], sems[bb]
                    ).wait()
                    pltpu.sync_copy(rows_v.at[bb], accum_sh.at[dst_v], add=True)

                    @pl.when(s + NB < S)
                    def _refill():
                        pltpu.async_copy(
                            table_hbm.at[idx_v.at[s + NB]], rows_v.at[bb],
                            sems[bb],
                        )
            return carry

        lax.fori_loop(0, (S - 1 + NB - 1) // NB, tail_body, 0)

        pltpu.sync_copy(accum_sh.at[pl.ds(lbase, BT)], acc_v)
        inv = jnp.full((L,), 1.0 / S, jnp.float32)

        def sbody(r, carry):
            for j in range(D // L):
                acc_v[r, pl.ds(j * L, L)] = acc_v[r, pl.ds(j * L, L)] * inv
            return carry

        lax.fori_loop(0, BT, sbody, 0)
        pltpu.sync_copy(acc_v, out_hbm.at[pl.ds(gbase, BT)])

    return k(sentence, table)


def kernel(sentence, table):
    return _mean_embed(sentence, table)


# speedup vs baseline: 13.3006x; 1.0041x over previous
import functools

import jax
import jax.numpy as jnp
from jax import lax
from jax.experimental import pallas as pl
from jax.experimental.pallas import tpu as pltpu
from jax.experimental.pallas import tpu_sc as plsc

VOCAB = 100000
D = 128
S = 200
B = 4096

NC = 2
NS = 16
L = 16
BT = B // (NC * NS)
SC_B = B // NC


def _mean_embed(sentence, table):
    mesh = plsc.VectorSubcoreMesh(core_axis_name="c", subcore_axis_name="s")

    @functools.partial(
        pl.kernel,
        mesh=mesh,
        out_type=jax.ShapeDtypeStruct((B, D), jnp.float32),
        scratch_types=[
            pltpu.VMEM((S, BT), jnp.int32),
            pltpu.VMEM((5, BT, D), jnp.float32),
            pltpu.VMEM((BT,), jnp.int32),
            pltpu.VMEM_SHARED((SC_B, D), jnp.float32),
            pltpu.SemaphoreType.DMA,
            pltpu.SemaphoreType.DMA,
            pltpu.SemaphoreType.DMA,
            pltpu.SemaphoreType.DMA,
            pltpu.SemaphoreType.DMA,
        ],
    )
    def k(sent_hbm, table_hbm, out_hbm, idx_v, rows_v, dst_v,
          accum_sh, sem0, sem1, sem2, sem3, sem4):
        cid = lax.axis_index("c")
        sid = lax.axis_index("s")
        tid = cid * NS + sid
        gbase = tid * BT
        lbase = sid * BT

        pltpu.sync_copy(sent_hbm.at[:, pl.ds(gbase, BT)], idx_v)

        for j in range(BT // L):
            dst_v[pl.ds(j * L, L)] = (
                jnp.full((L,), lbase + j * L, jnp.int32)
                + lax.iota(jnp.int32, L)
            )

        sems = (sem0, sem1, sem2, sem3, sem4)
        NB = 5

        for b in range(NB):
            pltpu.async_copy(table_hbm.at[idx_v.at[b]], rows_v.at[b], sems[b])

        pltpu.make_async_copy(
            table_hbm.at[idx_v.at[0]], rows_v.at[0], sems[0]
        ).wait()
        pltpu.sync_copy(rows_v.at[0], accum_sh.at[dst_v])
        pltpu.async_copy(table_hbm.at[idx_v.at[NB]], rows_v.at[0], sems[0])

        def tail_body(g, carry):
            for b in range(NB):
                s = NB * g + b + 1
                bb = (b + 1) % NB

                @pl.when(s < S)
                def _step():
                    pltpu.make_async_copy(
                        table_hbm.at[idx_v.at[0]], rows_v.at[bb], sems[bb]
                    ).wait()
                    pltpu.sync_copy(rows_v.at[bb], accum_sh.at[dst_v], add=True)

                    @pl.when(s + NB < S)
                    def _refill():
                        pltpu.async_copy(
                            table_hbm.at[idx_v.at[s + NB]], rows_v.at[bb],
                            sems[bb],
                        )
            return carry

        lax.fori_loop(0, (S - 1 + NB - 1) // NB, tail_body, 0)

        acc_v = rows_v.at[0]
        pltpu.sync_copy(accum_sh.at[pl.ds(lbase, BT)], acc_v)
        inv = jnp.full((L,), 1.0 / S, jnp.float32)

        def sbody(r, carry):
            for j in range(D // L):
                acc_v[r, pl.ds(j * L, L)] = acc_v[r, pl.ds(j * L, L)] * inv
            return carry

        lax.fori_loop(0, BT, sbody, 0)
        pltpu.sync_copy(acc_v, out_hbm.at[pl.ds(gbase, BT)])

    return k(sentence, table)


def kernel(sentence, table):
    return _mean_embed(sentence, table)
